# baseline (speedup 1.0000x reference)
import functools

import jax
import jax.numpy as jnp
from jax import lax
from jax.experimental import pallas as pl
from jax.experimental.pallas import tpu as pltpu
from jax.experimental.pallas import tpu_sc as plsc

N = 10000
NP = 10112
E = 160000
CH = 81
CW = 64
EP = 32 * CH * CW
RPT = NP // 16
DUMMY_DST = N + 1


def _pad64(g):
    return jnp.concatenate([g, jnp.zeros_like(g)], axis=1)


def _make_scatter(D):
    mesh = plsc.VectorSubcoreMesh(core_axis_name="c", subcore_axis_name="s")

    nbuf = 3

    @functools.partial(
        pl.kernel,
        mesh=mesh,
        out_type=jax.ShapeDtypeStruct((2, NP, D), jnp.float32),
        scratch_types=[
            pltpu.VMEM((CH, CW), jnp.int32),
            pltpu.VMEM((CH, CW), jnp.int32),
            pltpu.VMEM((nbuf, CW, D), jnp.float32),
            pltpu.VMEM_SHARED((NP, D), jnp.float32),
        ] + [pltpu.SemaphoreType.DMA] * (2 * nbuf),
    )
    def sck(g_hbm, src_hbm, dst_hbm, zeros_hbm, out_hbm, src_v, dst_v, rows_v, acc, *sems):
        gsem = sems[:nbuf]
        ssem = sems[nbuf:]
        c = lax.axis_index("c")
        s = lax.axis_index("s")
        wid = c * 16 + s
        base = s * RPT
        pltpu.sync_copy(src_hbm.at[wid], src_v)
        pltpu.sync_copy(dst_hbm.at[wid], dst_v)
        for b in range(2):
            pltpu.async_copy(g_hbm.at[src_v.at[b]], rows_v.at[b], gsem[b])
        pltpu.sync_copy(zeros_hbm, acc.at[pl.ds(base, RPT)])
        plsc.subcore_barrier()

        def body(t, carry):
            for b in range(nbuf):
                j = t * nbuf + b
                bp = (b + 2) % nbuf
                pltpu.make_async_copy(g_hbm.at[src_v.at[j]], rows_v.at[b], gsem[b]).wait()
                pltpu.async_copy(rows_v.at[b], acc.at[dst_v.at[j]], ssem[b], add=True)

                @pl.when(j >= 1)
                def _():
                    pltpu.make_async_copy(
                        rows_v.at[bp], acc.at[dst_v.at[j - 1]], ssem[bp]
                    ).wait()

                @pl.when(j + 2 < CH)
                def _():
                    pltpu.async_copy(g_hbm.at[src_v.at[j + 2]], rows_v.at[bp], gsem[bp])

            return carry

        lax.fori_loop(0, CH // nbuf, body, 0)
        bl = (CH - 1) % nbuf
        pltpu.make_async_copy(rows_v.at[bl], acc.at[dst_v.at[CH - 1]], ssem[bl]).wait()
        plsc.subcore_barrier()
        pltpu.sync_copy(acc.at[pl.ds(base, RPT)], out_hbm.at[c, pl.ds(base, RPT)])

    return sck


def _rowcall(fn, nodes, consts, out_dims, bm=400):
    grid = (N // bm,)
    in_specs = []
    for a in nodes:
        if a.ndim == 3:
            in_specs.append(pl.BlockSpec((2, bm, a.shape[2]), lambda i: (0, i, 0)))
        else:
            in_specs.append(pl.BlockSpec((bm, a.shape[1]), lambda i: (i, 0)))
    for a in consts:
        in_specs.append(pl.BlockSpec(a.shape, functools.partial(lambda nd, i: (0,) * nd, a.ndim)))

    nn, nc = len(nodes), len(consts)

    def body(*refs):
        ins = [r[...] for r in refs[: nn + nc]]
        outs = fn(*ins)
        if not isinstance(outs, tuple):
            outs = (outs,)
        for r, o in zip(refs[nn + nc :], outs):
            r[...] = o

    return pl.pallas_call(
        body,
        grid=grid,
        in_specs=in_specs,
        out_specs=[pl.BlockSpec((bm, d), lambda i: (i, 0)) for d in out_dims],
        out_shape=[jax.ShapeDtypeStruct((N, d), jnp.float32) for d in out_dims],
        compiler_params=pltpu.CompilerParams(dimension_semantics=("parallel",)),
    )(*nodes, *consts)


def _mm_big(Bm, W, b):
    bm = 400

    def body(x_ref, w_ref, b_ref, o_ref):
        o_ref[...] = jax.nn.relu(
            jnp.dot(x_ref[...], w_ref[...], preferred_element_type=jnp.float32) + b_ref[...]
        )

    return pl.pallas_call(
        body,
        grid=(N // bm,),
        in_specs=[
            pl.BlockSpec((bm, N), lambda i: (i, 0)),
            pl.BlockSpec((N, 256), lambda i: (0, 0)),
            pl.BlockSpec((1, 256), lambda i: (0, 0)),
        ],
        out_specs=pl.BlockSpec((bm, 256), lambda i: (i, 0)),
        out_shape=jax.ShapeDtypeStruct((N, 256), jnp.float32),
        compiler_params=pltpu.CompilerParams(dimension_semantics=("parallel",)),
    )(Bm, W, b)


def _mm_wide(xm, W, b):
    bm = 400
    K = xm.shape[1]

    def body(x_ref, w_ref, b_ref, o_ref):
        o_ref[...] = jax.nn.sigmoid(
            jnp.dot(x_ref[...], w_ref[...], preferred_element_type=jnp.float32) + b_ref[...]
        )

    return pl.pallas_call(
        body,
        grid=(N // bm,),
        in_specs=[
            pl.BlockSpec((bm, K), lambda i: (i, 0)),
            pl.BlockSpec((K, N), lambda i: (0, 0)),
            pl.BlockSpec((1, N), lambda i: (0, 0)),
        ],
        out_specs=pl.BlockSpec((bm, N), lambda i: (i, 0)),
        out_shape=jax.ShapeDtypeStruct((N, N), jnp.float32),
        compiler_params=pltpu.CompilerParams(dimension_semantics=("parallel",)),
    )(xm, W, b)


def _zzt(z):
    bm = 400

    def body(zi_ref, zj_ref, o_ref):
        d = lax.dot_general(
            zi_ref[...], zj_ref[...], (((1,), (1,)), ((), ())),
            preferred_element_type=jnp.float32,
        )
        o_ref[...] = jax.nn.sigmoid(d)

    return pl.pallas_call(
        body,
        grid=(N // bm,),
        in_specs=[
            pl.BlockSpec((bm, z.shape[1]), lambda i: (i, 0)),
            pl.BlockSpec((N, z.shape[1]), lambda i: (0, 0)),
        ],
        out_specs=pl.BlockSpec((bm, N), lambda i: (i, 0)),
        out_shape=jax.ShapeDtypeStruct((N, N), jnp.float32),
        compiler_params=pltpu.CompilerParams(dimension_semantics=("parallel",)),
    )(z, z)


def kernel(x, B, edge_index, params):
    p = params
    src = edge_index[0]
    dst = edge_index[1]
    pad = EP - E
    srcp = jnp.concatenate([src, jnp.zeros((pad,), jnp.int32)]).reshape(32, CH, CW)
    dstp = jnp.concatenate([dst, jnp.full((pad,), DUMMY_DST, jnp.int32)]).reshape(32, CH, CW)

    z128 = jnp.zeros((RPT, 128), jnp.float32)
    ones128 = jnp.ones((N, 128), jnp.float32)

    scat128 = _make_scatter(128)

    b2 = {k: v.reshape(1, -1) for k, v in p.items() if k.endswith('_b')}

    degp = scat128(ones128, srcp, dstp, z128)

    def f_norm(dg):
        d = dg[0, :, 0:1] + dg[1, :, 0:1]
        return jnp.broadcast_to(lax.rsqrt(jnp.maximum(d, 1.0)), (d.shape[0], 128))

    (norm,) = _rowcall(f_norm, [degp], [], [128])

    hidden1 = _mm_big(B, p['ce1_W'], b2['ce1_b'])

    def f_chain(h1, xx, nrm, ce2w, ce2b, ce3w, ce3b, cd1w, cd1b, cd2w, cd2b):
        h2 = jax.nn.relu(jnp.dot(h1, ce2w, preferred_element_type=jnp.float32) + ce2b)
        za = jax.nn.relu(jnp.dot(h2, ce3w, preferred_element_type=jnp.float32) + ce3b)
        s1 = jax.nn.relu(jnp.dot(za, cd1w, preferred_element_type=jnp.float32) + cd1b)
        s2 = jax.nn.relu(jnp.dot(s1, cd2w, preferred_element_type=jnp.float32) + cd2b)
        g1 = nrm[:, 0:1] * xx
        return h2, za, s1, s2, g1

    hidden2, z_a, se1, se2, g1 = _rowcall(
        f_chain,
        [hidden1, x, norm],
        [p['ce2_W'], b2['ce2_b'], p['ce3_W'], b2['ce3_b'],
         p['cd1_W'], b2['cd1_b'], p['cd2_W'], b2['cd2_b']],
        [128, 64, 128, 256, 128],
    )

    community_recon = _mm_wide(se2, p['cd3_W'], b2['cd3_b'])

    agg1 = scat128(g1, srcp, dstp, z128)

    def f_te1(agg, nrm, h1, w1, bb1, w2):
        n1 = nrm[:, 0:1]
        a = n1 * (agg[0] + agg[1])
        x1 = jax.nn.relu(jnp.dot(a, w1, preferred_element_type=jnp.float32) + bb1)
        h = x1 + h1
        return jnp.dot(n1 * h, w2, preferred_element_type=jnp.float32)

    (g2,) = _rowcall(f_te1, [agg1, norm, hidden1], [p['te1_W'], b2['te1_b'], p['te2_W']], [128])

    agg2 = scat128(g2, srcp, dstp, z128)

    def f_te2(agg, nrm, h2, bb, w):
        n1 = nrm[:, 0:1]
        x2 = jax.nn.relu(n1 * (agg[0] + agg[1]) + bb)
        return _pad64(jnp.dot(n1 * (x2 + h2), w, preferred_element_type=jnp.float32))

    (g3,) = _rowcall(f_te2, [agg2, norm, hidden2], [b2['te2_b'], p['te3_W']], [128])

    agg3 = scat128(g3, srcp, dstp, z128)

    def f_te3(agg, nrm, za, bb, w):
        n1 = nrm[:, 0:1]
        x3 = jax.nn.relu(n1 * (agg[0, :, :64] + agg[1, :, :64]) + bb)
        return _pad64(jnp.dot(n1 * (x3 + za), w, preferred_element_type=jnp.float32))

    (g4,) = _rowcall(f_te3, [agg3, norm, z_a], [b2['te3_b'], p['te4_W']], [128])

    agg4 = scat128(g4, srcp, dstp, z128)

    def f_te4(agg, nrm, bb, w):
        n1 = nrm[:, 0:1]
        z = jax.nn.relu(n1 * (agg[0, :, :64] + agg[1, :, :64]) + bb)
        return z, _pad64(jnp.dot(n1 * z, w, preferred_element_type=jnp.float32))

    z, g5 = _rowcall(f_te4, [agg4, norm], [b2['te4_b'], p['ad1_W']], [64, 128])

    agg5 = scat128(g5, srcp, dstp, z128)

    def f_ad1(agg, nrm, bb):
        n1 = nrm[:, 0:1]
        a1 = jax.nn.relu(n1 * (agg[0, :, :64] + agg[1, :, :64]) + bb)
        return _pad64(n1 * a1)

    (g6,) = _rowcall(f_ad1, [agg5, norm], [b2['ad1_b']], [128])

    agg6 = scat128(g6, srcp, dstp, z128)

    def f_ad2(agg, nrm, w, bb):
        n1 = nrm[:, 0:1]
        a2 = jax.nn.relu(jnp.dot(n1 * (agg[0, :, :64] + agg[1, :, :64]), w, preferred_element_type=jnp.float32) + bb)
        return n1 * a2

    (g7,) = _rowcall(f_ad2, [agg6, norm], [p['ad2_W'], b2['ad2_b']], [128])

    agg7 = scat128(g7, srcp, dstp, z128)

    def f_ad3(agg, nrm, w, bb, w4):
        n1 = nrm[:, 0:1]
        a3 = jax.nn.relu(jnp.dot(n1 * (agg[0] + agg[1]), w, preferred_element_type=jnp.float32) + bb)
        return jnp.dot(n1 * a3, w4, preferred_element_type=jnp.float32)

    (g8,) = _rowcall(f_ad3, [agg7, norm], [p['ad3_W'], b2['ad3_b'], p['ad4_W']], [128])

    agg8 = scat128(g8, srcp, dstp, z128)

    def f_ad4(agg, nrm, bb):
        n1 = nrm[:, 0:1]
        return jax.nn.relu(n1 * (agg[0] + agg[1]) + bb)

    (attr_recon,) = _rowcall(f_ad4, [agg8, norm], [b2['ad4_b']], [128])

    struct_recon = _zzt(z)
    return struct_recon, attr_recon, community_recon


# speedup vs baseline: 1.2176x; 1.2176x over previous
import functools

import jax
import jax.numpy as jnp
from jax import lax
from jax.experimental import pallas as pl
from jax.experimental.pallas import tpu as pltpu
from jax.experimental.pallas import tpu_sc as plsc

N = 10000
NP = 10112
E = 160000
CH = 40
CW = 128
EP = 32 * CH * CW
RPT = NP // 16
DUMMY_DST = N + 1


def _pad64(g):
    return jnp.concatenate([g, jnp.zeros_like(g)], axis=1)


def _make_scatter(D):
    mesh = plsc.VectorSubcoreMesh(core_axis_name="c", subcore_axis_name="s")

    nbuf = 2

    @functools.partial(
        pl.kernel,
        mesh=mesh,
        out_type=jax.ShapeDtypeStruct((2, NP, D), jnp.float32),
        scratch_types=[
            pltpu.VMEM((CH, CW), jnp.int32),
            pltpu.VMEM((CH, 2, CW // 2), jnp.int32),
            pltpu.VMEM((nbuf, CW, D), jnp.float32),
            pltpu.VMEM_SHARED((NP, D), jnp.float32),
        ] + [pltpu.SemaphoreType.DMA] * (nbuf + 2),
    )
    def sck(g_hbm, src_hbm, dst_hbm, zeros_hbm, out_hbm, src_v, dst_v, rows_v, acc, *sems):
        gsem = sems[:nbuf]
        ssem = sems[nbuf:]
        hw = CW // 2
        c = lax.axis_index("c")
        s = lax.axis_index("s")
        wid = c * 16 + s
        base = s * RPT
        pltpu.sync_copy(src_hbm.at[wid], src_v)
        pltpu.sync_copy(dst_hbm.at[wid], dst_v)
        for b in range(nbuf):
            pltpu.async_copy(g_hbm.at[src_v.at[b]], rows_v.at[b], gsem[b])
        pltpu.sync_copy(zeros_hbm, acc.at[pl.ds(base, RPT)])
        plsc.subcore_barrier()

        def body(t, carry):
            for b in range(nbuf):
                j = t * nbuf + b
                pltpu.make_async_copy(g_hbm.at[src_v.at[j]], rows_v.at[b], gsem[b]).wait()
                h0 = pltpu.async_copy(
                    rows_v.at[b, pl.ds(0, hw)], acc.at[dst_v.at[j, 0]], ssem[0], add=True
                )
                h1 = pltpu.async_copy(
                    rows_v.at[b, pl.ds(hw, hw)], acc.at[dst_v.at[j, 1]], ssem[1], add=True
                )
                h0.wait()
                h1.wait()
                nj = j + nbuf

                @pl.when(nj < CH)
                def _():
                    pltpu.async_copy(g_hbm.at[src_v.at[nj]], rows_v.at[b], gsem[b])

            return carry

        lax.fori_loop(0, CH // nbuf, body, 0)
        plsc.subcore_barrier()
        pltpu.sync_copy(acc.at[pl.ds(base, RPT)], out_hbm.at[c, pl.ds(base, RPT)])

    return sck


def _rowcall(fn, nodes, consts, out_dims, bm=400):
    grid = (N // bm,)
    in_specs = []
    for a in nodes:
        if a.ndim == 3:
            in_specs.append(pl.BlockSpec((2, bm, a.shape[2]), lambda i: (0, i, 0)))
        else:
            in_specs.append(pl.BlockSpec((bm, a.shape[1]), lambda i: (i, 0)))
    for a in consts:
        in_specs.append(pl.BlockSpec(a.shape, functools.partial(lambda nd, i: (0,) * nd, a.ndim)))

    nn, nc = len(nodes), len(consts)

    def body(*refs):
        ins = [r[...] for r in refs[: nn + nc]]
        outs = fn(*ins)
        if not isinstance(outs, tuple):
            outs = (outs,)
        for r, o in zip(refs[nn + nc :], outs):
            r[...] = o

    return pl.pallas_call(
        body,
        grid=grid,
        in_specs=in_specs,
        out_specs=[pl.BlockSpec((bm, d), lambda i: (i, 0)) for d in out_dims],
        out_shape=[jax.ShapeDtypeStruct((N, d), jnp.float32) for d in out_dims],
        compiler_params=pltpu.CompilerParams(dimension_semantics=("parallel",)),
    )(*nodes, *consts)


def _mm_big(Bm, W, b):
    bm = 400

    def body(x_ref, w_ref, b_ref, o_ref):
        o_ref[...] = jax.nn.relu(
            jnp.dot(x_ref[...], w_ref[...], preferred_element_type=jnp.float32) + b_ref[...]
        )

    return pl.pallas_call(
        body,
        grid=(N // bm,),
        in_specs=[
            pl.BlockSpec((bm, N), lambda i: (i, 0)),
            pl.BlockSpec((N, 256), lambda i: (0, 0)),
            pl.BlockSpec((1, 256), lambda i: (0, 0)),
        ],
        out_specs=pl.BlockSpec((bm, 256), lambda i: (i, 0)),
        out_shape=jax.ShapeDtypeStruct((N, 256), jnp.float32),
        compiler_params=pltpu.CompilerParams(dimension_semantics=("parallel",)),
    )(Bm, W, b)


def _mm_wide(xm, W, b):
    bm = 400
    K = xm.shape[1]

    def body(x_ref, w_ref, b_ref, o_ref):
        o_ref[...] = jax.nn.sigmoid(
            jnp.dot(x_ref[...], w_ref[...], preferred_element_type=jnp.float32) + b_ref[...]
        )

    return pl.pallas_call(
        body,
        grid=(N // bm,),
        in_specs=[
            pl.BlockSpec((bm, K), lambda i: (i, 0)),
            pl.BlockSpec((K, N), lambda i: (0, 0)),
            pl.BlockSpec((1, N), lambda i: (0, 0)),
        ],
        out_specs=pl.BlockSpec((bm, N), lambda i: (i, 0)),
        out_shape=jax.ShapeDtypeStruct((N, N), jnp.float32),
        compiler_params=pltpu.CompilerParams(dimension_semantics=("parallel",)),
    )(xm, W, b)


def _zzt(z):
    bm = 400

    def body(zi_ref, zj_ref, o_ref):
        d = lax.dot_general(
            zi_ref[...], zj_ref[...], (((1,), (1,)), ((), ())),
            preferred_element_type=jnp.float32,
        )
        o_ref[...] = jax.nn.sigmoid(d)

    return pl.pallas_call(
        body,
        grid=(N // bm,),
        in_specs=[
            pl.BlockSpec((bm, z.shape[1]), lambda i: (i, 0)),
            pl.BlockSpec((N, z.shape[1]), lambda i: (0, 0)),
        ],
        out_specs=pl.BlockSpec((bm, N), lambda i: (i, 0)),
        out_shape=jax.ShapeDtypeStruct((N, N), jnp.float32),
        compiler_params=pltpu.CompilerParams(dimension_semantics=("parallel",)),
    )(z, z)


def kernel(x, B, edge_index, params):
    p = params
    src = edge_index[0]
    dst = edge_index[1]
    pad = EP - E
    srcp = jnp.concatenate([src, jnp.zeros((pad,), jnp.int32)]).reshape(32, CH, CW)
    dstp = jnp.concatenate([dst, jnp.full((pad,), DUMMY_DST, jnp.int32)]).reshape(32, CH, 2, CW // 2)

    z128 = jnp.zeros((RPT, 128), jnp.float32)
    ones128 = jnp.ones((N, 128), jnp.float32)

    scat128 = _make_scatter(128)

    b2 = {k: v.reshape(1, -1) for k, v in p.items() if k.endswith('_b')}

    degp = scat128(ones128, srcp, dstp, z128)

    def f_norm(dg):
        d = dg[0, :, 0:1] + dg[1, :, 0:1]
        return jnp.broadcast_to(lax.rsqrt(jnp.maximum(d, 1.0)), (d.shape[0], 128))

    (norm,) = _rowcall(f_norm, [degp], [], [128])

    hidden1 = _mm_big(B, p['ce1_W'], b2['ce1_b'])

    def f_chain(h1, xx, nrm, ce2w, ce2b, ce3w, ce3b, cd1w, cd1b, cd2w, cd2b):
        h2 = jax.nn.relu(jnp.dot(h1, ce2w, preferred_element_type=jnp.float32) + ce2b)
        za = jax.nn.relu(jnp.dot(h2, ce3w, preferred_element_type=jnp.float32) + ce3b)
        s1 = jax.nn.relu(jnp.dot(za, cd1w, preferred_element_type=jnp.float32) + cd1b)
        s2 = jax.nn.relu(jnp.dot(s1, cd2w, preferred_element_type=jnp.float32) + cd2b)
        g1 = nrm[:, 0:1] * xx
        return h2, za, s1, s2, g1

    hidden2, z_a, se1, se2, g1 = _rowcall(
        f_chain,
        [hidden1, x, norm],
        [p['ce2_W'], b2['ce2_b'], p['ce3_W'], b2['ce3_b'],
         p['cd1_W'], b2['cd1_b'], p['cd2_W'], b2['cd2_b']],
        [128, 64, 128, 256, 128],
    )

    community_recon = _mm_wide(se2, p['cd3_W'], b2['cd3_b'])

    agg1 = scat128(g1, srcp, dstp, z128)

    def f_te1(agg, nrm, h1, w1, bb1, w2):
        n1 = nrm[:, 0:1]
        a = n1 * (agg[0] + agg[1])
        x1 = jax.nn.relu(jnp.dot(a, w1, preferred_element_type=jnp.float32) + bb1)
        h = x1 + h1
        return jnp.dot(n1 * h, w2, preferred_element_type=jnp.float32)

    (g2,) = _rowcall(f_te1, [agg1, norm, hidden1], [p['te1_W'], b2['te1_b'], p['te2_W']], [128])

    agg2 = scat128(g2, srcp, dstp, z128)

    def f_te2(agg, nrm, h2, bb, w):
        n1 = nrm[:, 0:1]
        x2 = jax.nn.relu(n1 * (agg[0] + agg[1]) + bb)
        return _pad64(jnp.dot(n1 * (x2 + h2), w, preferred_element_type=jnp.float32))

    (g3,) = _rowcall(f_te2, [agg2, norm, hidden2], [b2['te2_b'], p['te3_W']], [128])

    agg3 = scat128(g3, srcp, dstp, z128)

    def f_te3(agg, nrm, za, bb, w):
        n1 = nrm[:, 0:1]
        x3 = jax.nn.relu(n1 * (agg[0, :, :64] + agg[1, :, :64]) + bb)
        return _pad64(jnp.dot(n1 * (x3 + za), w, preferred_element_type=jnp.float32))

    (g4,) = _rowcall(f_te3, [agg3, norm, z_a], [b2['te3_b'], p['te4_W']], [128])

    agg4 = scat128(g4, srcp, dstp, z128)

    def f_te4(agg, nrm, bb, w):
        n1 = nrm[:, 0:1]
        z = jax.nn.relu(n1 * (agg[0, :, :64] + agg[1, :, :64]) + bb)
        return z, _pad64(jnp.dot(n1 * z, w, preferred_element_type=jnp.float32))

    z, g5 = _rowcall(f_te4, [agg4, norm], [b2['te4_b'], p['ad1_W']], [64, 128])

    agg5 = scat128(g5, srcp, dstp, z128)

    def f_ad1(agg, nrm, bb):
        n1 = nrm[:, 0:1]
        a1 = jax.nn.relu(n1 * (agg[0, :, :64] + agg[1, :, :64]) + bb)
        return _pad64(n1 * a1)

    (g6,) = _rowcall(f_ad1, [agg5, norm], [b2['ad1_b']], [128])

    agg6 = scat128(g6, srcp, dstp, z128)

    def f_ad2(agg, nrm, w, bb):
        n1 = nrm[:, 0:1]
        a2 = jax.nn.relu(jnp.dot(n1 * (agg[0, :, :64] + agg[1, :, :64]), w, preferred_element_type=jnp.float32) + bb)
        return n1 * a2

    (g7,) = _rowcall(f_ad2, [agg6, norm], [p['ad2_W'], b2['ad2_b']], [128])

    agg7 = scat128(g7, srcp, dstp, z128)

    def f_ad3(agg, nrm, w, bb, w4):
        n1 = nrm[:, 0:1]
        a3 = jax.nn.relu(jnp.dot(n1 * (agg[0] + agg[1]), w, preferred_element_type=jnp.float32) + bb)
        return jnp.dot(n1 * a3, w4, preferred_element_type=jnp.float32)

    (g8,) = _rowcall(f_ad3, [agg7, norm], [p['ad3_W'], b2['ad3_b'], p['ad4_W']], [128])

    agg8 = scat128(g8, srcp, dstp, z128)

    def f_ad4(agg, nrm, bb):
        n1 = nrm[:, 0:1]
        return jax.nn.relu(n1 * (agg[0] + agg[1]) + bb)

    (attr_recon,) = _rowcall(f_ad4, [agg8, norm], [b2['ad4_b']], [128])

    struct_recon = _zzt(z)
    return struct_recon, attr_recon, community_recon


# speedup vs baseline: 1.6689x; 1.3706x over previous
import functools

import jax
import jax.numpy as jnp
from jax import lax
from jax.experimental import pallas as pl
from jax.experimental.pallas import tpu as pltpu
from jax.experimental.pallas import tpu_sc as plsc

N = 10000
NP = 10112
E = 160000
CH = 40
CW = 128
EP = 32 * CH * CW
RPT = NP // 16
DUMMY_DST = N + 1


def _pad64(g):
    return jnp.concatenate([g, jnp.zeros_like(g)], axis=1)


def _make_scatter(D):
    mesh = plsc.VectorSubcoreMesh(core_axis_name="c", subcore_axis_name="s")

    nbuf = 2

    @functools.partial(
        pl.kernel,
        mesh=mesh,
        out_type=jax.ShapeDtypeStruct((2, NP, D), jnp.float32),
        scratch_types=[
            pltpu.VMEM((CH, CW), jnp.int32),
            pltpu.VMEM((CH, CW), jnp.int32),
            pltpu.VMEM((nbuf, CW, D), jnp.float32),
            pltpu.VMEM_SHARED((NP, D), jnp.float32),
        ] + [pltpu.SemaphoreType.DMA] * nbuf,
        compiler_params=pltpu.CompilerParams(use_tc_tiling_on_sc=False),
    )
    def sck(g_hbm, src_hbm, dst_hbm, zeros_hbm, out_hbm, src_v, dst_v, rows_v, acc, *sems):
        c = lax.axis_index("c")
        s = lax.axis_index("s")
        wid = c * 16 + s
        base = s * RPT
        pltpu.sync_copy(src_hbm.at[wid], src_v)
        pltpu.sync_copy(dst_hbm.at[wid], dst_v)
        for b in range(nbuf):
            pltpu.async_copy(g_hbm.at[src_v.at[b]], rows_v.at[b], sems[b])
        pltpu.sync_copy(zeros_hbm, acc.at[pl.ds(base, RPT)])
        plsc.subcore_barrier()

        def body(t, carry):
            for b in range(nbuf):
                j = t * nbuf + b
                pltpu.make_async_copy(g_hbm.at[src_v.at[j]], rows_v.at[b], sems[b]).wait()
                pltpu.sync_copy(rows_v.at[b], acc.at[dst_v.at[j]], add=True)
                nj = j + nbuf

                @pl.when(nj < CH)
                def _():
                    pltpu.async_copy(g_hbm.at[src_v.at[nj]], rows_v.at[b], sems[b])

            return carry

        lax.fori_loop(0, CH // nbuf, body, 0)
        plsc.subcore_barrier()
        pltpu.sync_copy(acc.at[pl.ds(base, RPT)], out_hbm.at[c, pl.ds(base, RPT)])

    return sck


def _rowcall(fn, nodes, consts, out_dims, bm=400):
    grid = (N // bm,)
    in_specs = []
    for a in nodes:
        if a.ndim == 3:
            in_specs.append(pl.BlockSpec((2, bm, a.shape[2]), lambda i: (0, i, 0)))
        else:
            in_specs.append(pl.BlockSpec((bm, a.shape[1]), lambda i: (i, 0)))
    for a in consts:
        in_specs.append(pl.BlockSpec(a.shape, functools.partial(lambda nd, i: (0,) * nd, a.ndim)))

    nn, nc = len(nodes), len(consts)

    def body(*refs):
        ins = [r[...] for r in refs[: nn + nc]]
        outs = fn(*ins)
        if not isinstance(outs, tuple):
            outs = (outs,)
        for r, o in zip(refs[nn + nc :], outs):
            r[...] = o

    return pl.pallas_call(
        body,
        grid=grid,
        in_specs=in_specs,
        out_specs=[pl.BlockSpec((bm, d), lambda i: (i, 0)) for d in out_dims],
        out_shape=[jax.ShapeDtypeStruct((N, d), jnp.float32) for d in out_dims],
        compiler_params=pltpu.CompilerParams(dimension_semantics=("parallel",)),
    )(*nodes, *consts)


def _mm_big(Bm, W, b):
    bm = 400

    def body(x_ref, w_ref, b_ref, o_ref):
        o_ref[...] = jax.nn.relu(
            jnp.dot(x_ref[...], w_ref[...], preferred_element_type=jnp.float32) + b_ref[...]
        )

    return pl.pallas_call(
        body,
        grid=(N // bm,),
        in_specs=[
            pl.BlockSpec((bm, N), lambda i: (i, 0)),
            pl.BlockSpec((N, 256), lambda i: (0, 0)),
            pl.BlockSpec((1, 256), lambda i: (0, 0)),
        ],
        out_specs=pl.BlockSpec((bm, 256), lambda i: (i, 0)),
        out_shape=jax.ShapeDtypeStruct((N, 256), jnp.float32),
        compiler_params=pltpu.CompilerParams(dimension_semantics=("parallel",)),
    )(Bm, W, b)


def _mm_wide(xm, W, b):
    bm = 400
    K = xm.shape[1]

    def body(x_ref, w_ref, b_ref, o_ref):
        o_ref[...] = jax.nn.sigmoid(
            jnp.dot(x_ref[...], w_ref[...], preferred_element_type=jnp.float32) + b_ref[...]
        )

    return pl.pallas_call(
        body,
        grid=(N // bm,),
        in_specs=[
            pl.BlockSpec((bm, K), lambda i: (i, 0)),
            pl.BlockSpec((K, N), lambda i: (0, 0)),
            pl.BlockSpec((1, N), lambda i: (0, 0)),
        ],
        out_specs=pl.BlockSpec((bm, N), lambda i: (i, 0)),
        out_shape=jax.ShapeDtypeStruct((N, N), jnp.float32),
        compiler_params=pltpu.CompilerParams(dimension_semantics=("parallel",)),
    )(xm, W, b)


def _zzt(z):
    bm = 400

    def body(zi_ref, zj_ref, o_ref):
        d = lax.dot_general(
            zi_ref[...], zj_ref[...], (((1,), (1,)), ((), ())),
            preferred_element_type=jnp.float32,
        )
        o_ref[...] = jax.nn.sigmoid(d)

    return pl.pallas_call(
        body,
        grid=(N // bm,),
        in_specs=[
            pl.BlockSpec((bm, z.shape[1]), lambda i: (i, 0)),
            pl.BlockSpec((N, z.shape[1]), lambda i: (0, 0)),
        ],
        out_specs=pl.BlockSpec((bm, N), lambda i: (i, 0)),
        out_shape=jax.ShapeDtypeStruct((N, N), jnp.float32),
        compiler_params=pltpu.CompilerParams(dimension_semantics=("parallel",)),
    )(z, z)


def kernel(x, B, edge_index, params):
    p = params
    src = edge_index[0]
    dst = edge_index[1]
    pad = EP - E
    srcp = jnp.concatenate([src, jnp.zeros((pad,), jnp.int32)]).reshape(32, CH, CW)
    dstp = jnp.concatenate([dst, jnp.full((pad,), DUMMY_DST, jnp.int32)]).reshape(32, CH, CW)

    z16 = jnp.zeros((RPT, 16), jnp.float32)
    z64 = jnp.zeros((RPT, 64), jnp.float32)
    z128 = jnp.zeros((RPT, 128), jnp.float32)
    ones16 = jnp.ones((N, 16), jnp.float32)

    scat16 = _make_scatter(16)
    scat64 = _make_scatter(64)
    scat128 = _make_scatter(128)

    b2 = {k: v.reshape(1, -1) for k, v in p.items() if k.endswith('_b')}

    degp = scat16(ones16, srcp, dstp, z16)

    def f_norm(dg):
        d = dg[0, :, 0:1] + dg[1, :, 0:1]
        return jnp.broadcast_to(lax.rsqrt(jnp.maximum(d, 1.0)), (d.shape[0], 128))

    (norm,) = _rowcall(f_norm, [degp], [], [128])

    hidden1 = _mm_big(B, p['ce1_W'], b2['ce1_b'])

    def f_chain(h1, xx, nrm, ce2w, ce2b, ce3w, ce3b, cd1w, cd1b, cd2w, cd2b):
        h2 = jax.nn.relu(jnp.dot(h1, ce2w, preferred_element_type=jnp.float32) + ce2b)
        za = jax.nn.relu(jnp.dot(h2, ce3w, preferred_element_type=jnp.float32) + ce3b)
        s1 = jax.nn.relu(jnp.dot(za, cd1w, preferred_element_type=jnp.float32) + cd1b)
        s2 = jax.nn.relu(jnp.dot(s1, cd2w, preferred_element_type=jnp.float32) + cd2b)
        g1 = nrm[:, 0:1] * xx
        return h2, za, s1, s2, g1

    hidden2, z_a, se1, se2, g1 = _rowcall(
        f_chain,
        [hidden1, x, norm],
        [p['ce2_W'], b2['ce2_b'], p['ce3_W'], b2['ce3_b'],
         p['cd1_W'], b2['cd1_b'], p['cd2_W'], b2['cd2_b']],
        [128, 64, 128, 256, 128],
    )

    community_recon = _mm_wide(se2, p['cd3_W'], b2['cd3_b'])

    agg1 = scat128(g1, srcp, dstp, z128)

    def f_te1(agg, nrm, h1, w1, bb1, w2):
        n1 = nrm[:, 0:1]
        a = n1 * (agg[0] + agg[1])
        x1 = jax.nn.relu(jnp.dot(a, w1, preferred_element_type=jnp.float32) + bb1)
        h = x1 + h1
        return jnp.dot(n1 * h, w2, preferred_element_type=jnp.float32)

    (g2,) = _rowcall(f_te1, [agg1, norm, hidden1], [p['te1_W'], b2['te1_b'], p['te2_W']], [128])

    agg2 = scat128(g2, srcp, dstp, z128)

    def f_te2(agg, nrm, h2, bb, w):
        n1 = nrm[:, 0:1]
        x2 = jax.nn.relu(n1 * (agg[0] + agg[1]) + bb)
        return jnp.dot(n1 * (x2 + h2), w, preferred_element_type=jnp.float32)

    (g3,) = _rowcall(f_te2, [agg2, norm, hidden2], [b2['te2_b'], p['te3_W']], [64])

    agg3 = scat64(g3, srcp, dstp, z64)

    def f_te3(agg, nrm, za, bb, w):
        n1 = nrm[:, 0:1]
        x3 = jax.nn.relu(n1 * (agg[0] + agg[1]) + bb)
        return jnp.dot(n1 * (x3 + za), w, preferred_element_type=jnp.float32)

    (g4,) = _rowcall(f_te3, [agg3, norm, z_a], [b2['te3_b'], p['te4_W']], [64])

    agg4 = scat64(g4, srcp, dstp, z64)

    def f_te4(agg, nrm, bb, w):
        n1 = nrm[:, 0:1]
        z = jax.nn.relu(n1 * (agg[0] + agg[1]) + bb)
        return z, jnp.dot(n1 * z, w, preferred_element_type=jnp.float32)

    z, g5 = _rowcall(f_te4, [agg4, norm], [b2['te4_b'], p['ad1_W']], [64, 64])

    agg5 = scat64(g5, srcp, dstp, z64)

    def f_ad1(agg, nrm, bb):
        n1 = nrm[:, 0:1]
        a1 = jax.nn.relu(n1 * (agg[0] + agg[1]) + bb)
        return n1 * a1

    (g6,) = _rowcall(f_ad1, [agg5, norm], [b2['ad1_b']], [64])

    agg6 = scat64(g6, srcp, dstp, z64)

    def f_ad2(agg, nrm, w, bb):
        n1 = nrm[:, 0:1]
        a2 = jax.nn.relu(jnp.dot(n1 * (agg[0] + agg[1]), w, preferred_element_type=jnp.float32) + bb)
        return n1 * a2

    (g7,) = _rowcall(f_ad2, [agg6, norm], [p['ad2_W'], b2['ad2_b']], [128])

    agg7 = scat128(g7, srcp, dstp, z128)

    def f_ad3(agg, nrm, w, bb, w4):
        n1 = nrm[:, 0:1]
        a3 = jax.nn.relu(jnp.dot(n1 * (agg[0] + agg[1]), w, preferred_element_type=jnp.float32) + bb)
        return jnp.dot(n1 * a3, w4, preferred_element_type=jnp.float32)

    (g8,) = _rowcall(f_ad3, [agg7, norm], [p['ad3_W'], b2['ad3_b'], p['ad4_W']], [128])

    agg8 = scat128(g8, srcp, dstp, z128)

    def f_ad4(agg, nrm, bb):
        n1 = nrm[:, 0:1]
        return jax.nn.relu(n1 * (agg[0] + agg[1]) + bb)

    (attr_recon,) = _rowcall(f_ad4, [agg8, norm], [b2['ad4_b']], [128])

    struct_recon = _zzt(z)
    return struct_recon, attr_recon, community_recon


# speedup vs baseline: 1.6972x; 1.0170x over previous
import functools

import jax
import jax.numpy as jnp
from jax import lax
from jax.experimental import pallas as pl
from jax.experimental.pallas import tpu as pltpu
from jax.experimental.pallas import tpu_sc as plsc

N = 10000
NP = 10112
E = 160000
CH = 40
CW = 128
EP = 32 * CH * CW
RPT = NP // 16
DUMMY_DST = N + 1


def _pad64(g):
    return jnp.concatenate([g, jnp.zeros_like(g)], axis=1)


def _make_scatter(D, tiled=False):
    mesh = plsc.VectorSubcoreMesh(core_axis_name="c", subcore_axis_name="s")

    nbuf = 2

    @functools.partial(
        pl.kernel,
        mesh=mesh,
        out_type=jax.ShapeDtypeStruct((2, NP, D), jnp.float32),
        scratch_types=[
            pltpu.VMEM((CH, CW), jnp.int32),
            pltpu.VMEM((CH, CW), jnp.int32),
            pltpu.VMEM((nbuf, CW, D), jnp.float32),
            pltpu.VMEM_SHARED((NP, D), jnp.float32),
        ] + [pltpu.SemaphoreType.DMA] * nbuf,
        compiler_params=pltpu.CompilerParams(use_tc_tiling_on_sc=tiled),
    )
    def sck(g_hbm, src_hbm, dst_hbm, zeros_hbm, out_hbm, src_v, dst_v, rows_v, acc, *sems):
        c = lax.axis_index("c")
        s = lax.axis_index("s")
        wid = c * 16 + s
        base = s * RPT
        pltpu.sync_copy(src_hbm.at[wid], src_v)
        pltpu.sync_copy(dst_hbm.at[wid], dst_v)
        for b in range(nbuf):
            pltpu.async_copy(g_hbm.at[src_v.at[b]], rows_v.at[b], sems[b])
        pltpu.sync_copy(zeros_hbm, acc.at[pl.ds(base, RPT)])
        plsc.subcore_barrier()

        def body(t, carry):
            for b in range(nbuf):
                j = t * nbuf + b
                pltpu.make_async_copy(g_hbm.at[src_v.at[j]], rows_v.at[b], sems[b]).wait()
                pltpu.sync_copy(rows_v.at[b], acc.at[dst_v.at[j]], add=True)
                nj = j + nbuf

                @pl.when(nj < CH)
                def _():
                    pltpu.async_copy(g_hbm.at[src_v.at[nj]], rows_v.at[b], sems[b])

            return carry

        lax.fori_loop(0, CH // nbuf, body, 0)
        plsc.subcore_barrier()
        pltpu.sync_copy(acc.at[pl.ds(base, RPT)], out_hbm.at[c, pl.ds(base, RPT)])

    return sck


def _rowcall(fn, nodes, consts, out_dims, bm=400):
    grid = (N // bm,)
    in_specs = []
    for a in nodes:
        if a.ndim == 3:
            in_specs.append(pl.BlockSpec((2, bm, a.shape[2]), lambda i: (0, i, 0)))
        else:
            in_specs.append(pl.BlockSpec((bm, a.shape[1]), lambda i: (i, 0)))
    for a in consts:
        in_specs.append(pl.BlockSpec(a.shape, functools.partial(lambda nd, i: (0,) * nd, a.ndim)))

    nn, nc = len(nodes), len(consts)

    def body(*refs):
        ins = [r[...] for r in refs[: nn + nc]]
        outs = fn(*ins)
        if not isinstance(outs, tuple):
            outs = (outs,)
        for r, o in zip(refs[nn + nc :], outs):
            r[...] = o

    return pl.pallas_call(
        body,
        grid=grid,
        in_specs=in_specs,
        out_specs=[pl.BlockSpec((bm, d), lambda i: (i, 0)) for d in out_dims],
        out_shape=[jax.ShapeDtypeStruct((N, d), jnp.float32) for d in out_dims],
        compiler_params=pltpu.CompilerParams(dimension_semantics=("parallel",)),
    )(*nodes, *consts)


def _mm_big(Bm, W, b):
    bm = 400

    def body(x_ref, w_ref, b_ref, o_ref):
        o_ref[...] = jax.nn.relu(
            jnp.dot(x_ref[...], w_ref[...], preferred_element_type=jnp.float32) + b_ref[...]
        )

    return pl.pallas_call(
        body,
        grid=(N // bm,),
        in_specs=[
            pl.BlockSpec((bm, N), lambda i: (i, 0)),
            pl.BlockSpec((N, 256), lambda i: (0, 0)),
            pl.BlockSpec((1, 256), lambda i: (0, 0)),
        ],
        out_specs=pl.BlockSpec((bm, 256), lambda i: (i, 0)),
        out_shape=jax.ShapeDtypeStruct((N, 256), jnp.float32),
        compiler_params=pltpu.CompilerParams(dimension_semantics=("parallel",)),
    )(Bm, W, b)


def _mm_wide(xm, W, b):
    bm = 400
    K = xm.shape[1]

    def body(x_ref, w_ref, b_ref, o_ref):
        o_ref[...] = jax.nn.sigmoid(
            jnp.dot(x_ref[...], w_ref[...], preferred_element_type=jnp.float32) + b_ref[...]
        )

    return pl.pallas_call(
        body,
        grid=(N // bm,),
        in_specs=[
            pl.BlockSpec((bm, K), lambda i: (i, 0)),
            pl.BlockSpec((K, N), lambda i: (0, 0)),
            pl.BlockSpec((1, N), lambda i: (0, 0)),
        ],
        out_specs=pl.BlockSpec((bm, N), lambda i: (i, 0)),
        out_shape=jax.ShapeDtypeStruct((N, N), jnp.float32),
        compiler_params=pltpu.CompilerParams(dimension_semantics=("parallel",)),
    )(xm, W, b)


def _zzt(z):
    bm = 400

    def body(zi_ref, zj_ref, o_ref):
        d = lax.dot_general(
            zi_ref[...], zj_ref[...], (((1,), (1,)), ((), ())),
            preferred_element_type=jnp.float32,
        )
        o_ref[...] = jax.nn.sigmoid(d)

    return pl.pallas_call(
        body,
        grid=(N // bm,),
        in_specs=[
            pl.BlockSpec((bm, z.shape[1]), lambda i: (i, 0)),
            pl.BlockSpec((N, z.shape[1]), lambda i: (0, 0)),
        ],
        out_specs=pl.BlockSpec((bm, N), lambda i: (i, 0)),
        out_shape=jax.ShapeDtypeStruct((N, N), jnp.float32),
        compiler_params=pltpu.CompilerParams(dimension_semantics=("parallel",)),
    )(z, z)


def kernel(x, B, edge_index, params):
    p = params
    src = edge_index[0]
    dst = edge_index[1]
    pad = EP - E
    srcp = jnp.concatenate([src, jnp.zeros((pad,), jnp.int32)]).reshape(32, CH, CW)
    dstp = jnp.concatenate([dst, jnp.full((pad,), DUMMY_DST, jnp.int32)]).reshape(32, CH, CW)

    z16 = jnp.zeros((RPT, 16), jnp.float32)
    z64 = jnp.zeros((RPT, 64), jnp.float32)
    z128 = jnp.zeros((RPT, 128), jnp.float32)
    ones16 = jnp.ones((N, 16), jnp.float32)

    scat16 = _make_scatter(16)
    scat64 = _make_scatter(64)
    scat128 = _make_scatter(128, tiled=True)

    b2 = {k: v.reshape(1, -1) for k, v in p.items() if k.endswith('_b')}

    degp = scat16(ones16, srcp, dstp, z16)

    def f_norm(dg):
        d = dg[0, :, 0:1] + dg[1, :, 0:1]
        return jnp.broadcast_to(lax.rsqrt(jnp.maximum(d, 1.0)), (d.shape[0], 128))

    (norm,) = _rowcall(f_norm, [degp], [], [128])

    hidden1 = _mm_big(B, p['ce1_W'], b2['ce1_b'])

    def f_chain(h1, xx, nrm, ce2w, ce2b, ce3w, ce3b, cd1w, cd1b, cd2w, cd2b):
        h2 = jax.nn.relu(jnp.dot(h1, ce2w, preferred_element_type=jnp.float32) + ce2b)
        za = jax.nn.relu(jnp.dot(h2, ce3w, preferred_element_type=jnp.float32) + ce3b)
        s1 = jax.nn.relu(jnp.dot(za, cd1w, preferred_element_type=jnp.float32) + cd1b)
        s2 = jax.nn.relu(jnp.dot(s1, cd2w, preferred_element_type=jnp.float32) + cd2b)
        g1 = nrm[:, 0:1] * xx
        return h2, za, s1, s2, g1

    hidden2, z_a, se1, se2, g1 = _rowcall(
        f_chain,
        [hidden1, x, norm],
        [p['ce2_W'], b2['ce2_b'], p['ce3_W'], b2['ce3_b'],
         p['cd1_W'], b2['cd1_b'], p['cd2_W'], b2['cd2_b']],
        [128, 64, 128, 256, 128],
    )

    community_recon = _mm_wide(se2, p['cd3_W'], b2['cd3_b'])

    agg1 = scat128(g1, srcp, dstp, z128)

    def f_te1(agg, nrm, h1, w1, bb1, w2):
        n1 = nrm[:, 0:1]
        a = n1 * (agg[0] + agg[1])
        x1 = jax.nn.relu(jnp.dot(a, w1, preferred_element_type=jnp.float32) + bb1)
        h = x1 + h1
        return jnp.dot(n1 * h, w2, preferred_element_type=jnp.float32)

    (g2,) = _rowcall(f_te1, [agg1, norm, hidden1], [p['te1_W'], b2['te1_b'], p['te2_W']], [128])

    agg2 = scat128(g2, srcp, dstp, z128)

    def f_te2(agg, nrm, h2, bb, w):
        n1 = nrm[:, 0:1]
        x2 = jax.nn.relu(n1 * (agg[0] + agg[1]) + bb)
        return jnp.dot(n1 * (x2 + h2), w, preferred_element_type=jnp.float32)

    (g3,) = _rowcall(f_te2, [agg2, norm, hidden2], [b2['te2_b'], p['te3_W']], [64])

    agg3 = scat64(g3, srcp, dstp, z64)

    def f_te3(agg, nrm, za, bb, w):
        n1 = nrm[:, 0:1]
        x3 = jax.nn.relu(n1 * (agg[0] + agg[1]) + bb)
        return jnp.dot(n1 * (x3 + za), w, preferred_element_type=jnp.float32)

    (g4,) = _rowcall(f_te3, [agg3, norm, z_a], [b2['te3_b'], p['te4_W']], [64])

    agg4 = scat64(g4, srcp, dstp, z64)

    def f_te4(agg, nrm, bb, w):
        n1 = nrm[:, 0:1]
        z = jax.nn.relu(n1 * (agg[0] + agg[1]) + bb)
        return z, jnp.dot(n1 * z, w, preferred_element_type=jnp.float32)

    z, g5 = _rowcall(f_te4, [agg4, norm], [b2['te4_b'], p['ad1_W']], [64, 64])

    agg5 = scat64(g5, srcp, dstp, z64)

    def f_ad1(agg, nrm, bb):
        n1 = nrm[:, 0:1]
        a1 = jax.nn.relu(n1 * (agg[0] + agg[1]) + bb)
        return n1 * a1

    (g6,) = _rowcall(f_ad1, [agg5, norm], [b2['ad1_b']], [64])

    agg6 = scat64(g6, srcp, dstp, z64)

    def f_ad2(agg, nrm, w, bb):
        n1 = nrm[:, 0:1]
        a2 = jax.nn.relu(jnp.dot(n1 * (agg[0] + agg[1]), w, preferred_element_type=jnp.float32) + bb)
        return n1 * a2

    (g7,) = _rowcall(f_ad2, [agg6, norm], [p['ad2_W'], b2['ad2_b']], [128])

    agg7 = scat128(g7, srcp, dstp, z128)

    def f_ad3(agg, nrm, w, bb, w4):
        n1 = nrm[:, 0:1]
        a3 = jax.nn.relu(jnp.dot(n1 * (agg[0] + agg[1]), w, preferred_element_type=jnp.float32) + bb)
        return jnp.dot(n1 * a3, w4, preferred_element_type=jnp.float32)

    (g8,) = _rowcall(f_ad3, [agg7, norm], [p['ad3_W'], b2['ad3_b'], p['ad4_W']], [128])

    agg8 = scat128(g8, srcp, dstp, z128)

    def f_ad4(agg, nrm, bb):
        n1 = nrm[:, 0:1]
        return jax.nn.relu(n1 * (agg[0] + agg[1]) + bb)

    (attr_recon,) = _rowcall(f_ad4, [agg8, norm], [b2['ad4_b']], [128])

    struct_recon = _zzt(z)
    return struct_recon, attr_recon, community_recon
